# baseline (speedup 1.0000x reference)
import jax
import jax.numpy as jnp
from jax import lax
from jax.experimental import pallas as pl
from jax.experimental.pallas import tpu as pltpu
from jax.experimental.pallas import tpu_sc as plsc

B, T_IN, N, F_IN = 2, 12, 10000, 128
H, GH, T_OUT, E = 128, 128, 24, 160000
BTN = B * T_IN * N

NC, NS, LANES = 2, 16, 16
EPT = E // NS
CHUNK = 128
ROWS = 80
NACC = 10112
SPAN = 632


def _sc_mesh():
    return plsc.VectorSubcoreMesh(core_axis_name="c", subcore_axis_name="s")


HALF = ROWS // 2


def _spmm_body(tbl, srcb, dstb, out, accum, dst_v, srcq, gb0, gb1,
               sem0, sem1, ssem0, ssem1):
    c = lax.axis_index("c")
    s = lax.axis_index("s")

    pltpu.sync_copy(dstb.at[s], dst_v)

    def zero_span():
        @pl.loop(0, CHUNK)
        def _(r):
            for k in range(F_IN // LANES):
                gb0[r, pl.ds(k * LANES, LANES)] = jnp.zeros((LANES,),
                                                            jnp.float32)

        for q in range(SPAN // CHUNK):
            pltpu.sync_copy(gb0, accum.at[pl.ds(s * SPAN + q * CHUNK, CHUNK)])
        _rem = SPAN - (SPAN // CHUNK) * CHUNK
        pltpu.sync_copy(gb0.at[pl.ds(0, _rem)],
                        accum.at[pl.ds(s * SPAN + SPAN - _rem, _rem)])

    zero_span()
    plsc.subcore_barrier()

    @pl.loop(0, T_IN)
    def _(t):
        off = (c * T_IN + t) * N

        for half in range(2):
            h0 = half * HALF
            pltpu.sync_copy(srcb.at[s, pl.ds(h0, HALF)], srcq)

            @pl.loop(0, HALF)
            def _(r):
                for k in range(CHUNK // LANES):
                    sl = pl.ds(k * LANES, LANES)
                    srcq[r, sl] = srcq[r, sl] + off

            bufs = ((gb0, sem0, ssem0), (gb1, sem1, ssem1))
            pltpu.async_copy(tbl.at[srcq.at[0]], gb0, sem0)

            @pl.loop(0, HALF, step=2)
            def _(j):
                for b in range(2):
                    gbuf, gsem, ssem = bufs[b]
                    obuf, ogsem, ossem = bufs[1 - b]
                    ch = j + b
                    pltpu.make_async_copy(tbl.at[srcq.at[ch]], gbuf,
                                          gsem).wait()
                    pltpu.async_copy(gbuf, accum.at[dst_v.at[h0 + ch]],
                                     ssem, add=True)

                    @pl.when(ch + 1 < HALF)
                    def _():
                        @pl.when(ch >= 1)
                        def _():
                            pltpu.make_async_copy(
                                obuf, accum.at[dst_v.at[h0 + ch - 1]],
                                ossem).wait()

                        pltpu.async_copy(tbl.at[srcq.at[ch + 1]], obuf,
                                         ogsem)

            for ch in (HALF - 2, HALF - 1):
                gbuf, _, ssem = bufs[ch % 2]
                pltpu.make_async_copy(gbuf, accum.at[dst_v.at[h0 + ch]],
                                      ssem).wait()

        plsc.subcore_barrier()

        pltpu.sync_copy(accum.at[pl.ds(s * SPAN, 520)],
                        out.at[pl.ds(off + s * SPAN, 520)])

        @pl.when(s < NS - 1)
        def _():
            pltpu.sync_copy(accum.at[pl.ds(s * SPAN + 520, 112)],
                            out.at[pl.ds(off + s * SPAN + 520, 112)])

        zero_span()
        plsc.subcore_barrier()


def _sc_spmm(tbl, srcb, dstb):
    return pl.kernel(
        _spmm_body,
        out_type=jax.ShapeDtypeStruct((BTN, F_IN), jnp.float32),
        mesh=_sc_mesh(),
        scratch_types=[
            pltpu.VMEM_SHARED((NACC, F_IN), jnp.float32),
            pltpu.VMEM((ROWS, CHUNK), jnp.int32),
            pltpu.VMEM((HALF, CHUNK), jnp.int32),
            pltpu.VMEM((CHUNK, F_IN), jnp.float32),
            pltpu.VMEM((CHUNK, F_IN), jnp.float32),
            pltpu.SemaphoreType.DMA,
            pltpu.SemaphoreType.DMA,
            pltpu.SemaphoreType.DMA,
            pltpu.SemaphoreType.DMA,
        ],
    )(tbl, srcb, dstb)


def _counts_body(dstb, out, cnt_acc, dst_v, buf):
    c = lax.axis_index("c")
    s = lax.axis_index("s")

    @pl.when(c == 0)
    def _():
        def fill(val):
            @pl.loop(0, CHUNK)
            def _(r):
                for k in range(F_IN // LANES):
                    buf[r, pl.ds(k * LANES, LANES)] = jnp.full(
                        (LANES,), val, jnp.float32)

        pltpu.sync_copy(dstb.at[s], dst_v)
        fill(0.0)
        for q in range(SPAN // CHUNK):
            pltpu.sync_copy(buf, cnt_acc.at[pl.ds(s * SPAN + q * CHUNK,
                                                  CHUNK)])
        _rem = SPAN - (SPAN // CHUNK) * CHUNK
        pltpu.sync_copy(buf.at[pl.ds(0, _rem)],
                        cnt_acc.at[pl.ds(s * SPAN + SPAN - _rem, _rem)])
        fill(1.0)
        plsc.subcore_barrier()

        @pl.loop(0, ROWS)
        def _(j):
            pltpu.sync_copy(buf, cnt_acc.at[dst_v.at[j]], add=True)

        plsc.subcore_barrier()
        pltpu.sync_copy(cnt_acc.at[pl.ds(s * SPAN, 520)],
                        out.at[pl.ds(s * SPAN, 520)])

        @pl.when(s < NS - 1)
        def _():
            pltpu.sync_copy(cnt_acc.at[pl.ds(s * SPAN + 520, 112)],
                            out.at[pl.ds(s * SPAN + 520, 112)])


def _sc_counts(dstb):
    return pl.kernel(
        _counts_body,
        out_type=jax.ShapeDtypeStruct((N, F_IN), jnp.float32),
        mesh=_sc_mesh(),
        scratch_types=[
            pltpu.VMEM_SHARED((NACC, F_IN), jnp.float32),
            pltpu.VMEM((ROWS, CHUNK), jnp.int32),
            pltpu.VMEM((CHUNK, F_IN), jnp.float32),
        ],
    )(dstb)


ROWBLK = 1000


def _combine_body(agg_ref, x_ref, cnt_ref, wl_ref, wr_ref, b_ref, o_ref):
    recip = 1.0 / jnp.maximum(cnt_ref[:, 0:1], 1.0)
    agg = agg_ref[...] * recip
    acc = jnp.dot(agg, wl_ref[...], preferred_element_type=jnp.float32)
    acc += jnp.dot(x_ref[...], wr_ref[...], preferred_element_type=jnp.float32)
    o_ref[...] = jnp.maximum(acc + b_ref[...], 0.0)


def _tc_combine(agg, x, cnt, wl_t, wr_t, bias):
    nb = N // ROWBLK
    return pl.pallas_call(
        _combine_body,
        grid=(B * T_IN, nb),
        in_specs=[
            pl.BlockSpec((ROWBLK, F_IN), lambda i, j: (i * nb + j, 0)),
            pl.BlockSpec((ROWBLK, F_IN), lambda i, j: (i * nb + j, 0)),
            pl.BlockSpec((ROWBLK, F_IN), lambda i, j: (j, 0)),
            pl.BlockSpec((F_IN, H), lambda i, j: (0, 0)),
            pl.BlockSpec((F_IN, H), lambda i, j: (0, 0)),
            pl.BlockSpec((1, H), lambda i, j: (0, 0)),
        ],
        out_specs=pl.BlockSpec((ROWBLK, H), lambda i, j: (i * nb + j, 0)),
        out_shape=jax.ShapeDtypeStruct((BTN, H), jnp.float32),
    )(agg, x, cnt, wl_t, wr_t, bias)


GRUBLK = 2000


def _gru_body(h1_ref, wih_ref, bih_ref, whh_ref, bhh_ref, w1_ref, b1_ref,
              w2_ref, b2_ref, o_ref, h_s):
    t = pl.program_id(1)

    @pl.when(t == 0)
    def _():
        h_s[...] = jnp.zeros((GRUBLK, GH), jnp.float32)

    xt = h1_ref[0, 0]
    h = h_s[...]
    gi = jnp.dot(xt, wih_ref[...], preferred_element_type=jnp.float32)
    gi += bih_ref[...]
    gh = jnp.dot(h, whh_ref[...], preferred_element_type=jnp.float32)
    gh += bhh_ref[...]
    r = jax.nn.sigmoid(gi[:, 0:GH] + gh[:, 0:GH])
    z = jax.nn.sigmoid(gi[:, GH:2 * GH] + gh[:, GH:2 * GH])
    n = jnp.tanh(gi[:, 2 * GH:] + r * gh[:, 2 * GH:])
    h = (1.0 - z) * n + z * h
    h_s[...] = h

    @pl.when(t == T_IN - 1)
    def _():
        h1 = jnp.maximum(
            jnp.dot(h, w1_ref[...], preferred_element_type=jnp.float32)
            + b1_ref[...], 0.0)
        o_ref[...] = (
            jnp.dot(h1, w2_ref[...], preferred_element_type=jnp.float32)
            + b2_ref[...])


def _tc_gru(h1_4d, wih_t, b_ih, whh_t, b_hh, w1_t, b1, w2_t, b2):
    nb = N // GRUBLK
    return pl.pallas_call(
        _gru_body,
        grid=(B * nb, T_IN),
        in_specs=[
            pl.BlockSpec((1, 1, GRUBLK, H),
                         lambda r, t: (r // nb, t, r % nb, 0)),
            pl.BlockSpec((H, 3 * GH), lambda r, t: (0, 0)),
            pl.BlockSpec((1, 3 * GH), lambda r, t: (0, 0)),
            pl.BlockSpec((GH, 3 * GH), lambda r, t: (0, 0)),
            pl.BlockSpec((1, 3 * GH), lambda r, t: (0, 0)),
            pl.BlockSpec((GH, GH), lambda r, t: (0, 0)),
            pl.BlockSpec((1, GH), lambda r, t: (0, 0)),
            pl.BlockSpec((GH, T_OUT), lambda r, t: (0, 0)),
            pl.BlockSpec((1, T_OUT), lambda r, t: (0, 0)),
        ],
        out_specs=pl.BlockSpec((GRUBLK, T_OUT), lambda r, t: (r, 0)),
        out_shape=jax.ShapeDtypeStruct((B * N, T_OUT), jnp.float32),
        scratch_shapes=[pltpu.VMEM((GRUBLK, GH), jnp.float32)],
    )(h1_4d, wih_t, b_ih, whh_t, b_hh, w1_t, b1, w2_t, b2)


def kernel(x_seq, edge_index, Wl0, bl0, Wr0, Wl1, bl1, Wr1, W_ih, W_hh,
           b_ih, b_hh, W1, b1, W2, b2):
    src = edge_index[0].astype(jnp.int32)
    dst = edge_index[1].astype(jnp.int32)
    srcb = jnp.pad(src.reshape(NS, EPT), ((0, 0), (0, ROWS * CHUNK - EPT)),
                   constant_values=0).reshape(NS, ROWS, CHUNK)
    dstb = jnp.pad(dst.reshape(NS, EPT), ((0, 0), (0, ROWS * CHUNK - EPT)),
                   constant_values=N).reshape(NS, ROWS, CHUNK)

    x_flat = x_seq.reshape(BTN, F_IN)
    cnt = _sc_counts(dstb)

    agg0 = _sc_spmm(x_flat, srcb, dstb)
    h0 = _tc_combine(agg0, x_flat, cnt, Wl0.T, Wr0.T, bl0.reshape(1, H))
    agg1 = _sc_spmm(h0, srcb, dstb)
    h1 = _tc_combine(agg1, h0, cnt, Wl1.T, Wr1.T, bl1.reshape(1, H))

    out_flat = _tc_gru(h1.reshape(B, T_IN, N, H), W_ih.T,
                       b_ih.reshape(1, 3 * GH), W_hh.T,
                       b_hh.reshape(1, 3 * GH), W1.T, b1.reshape(1, GH),
                       W2.T, b2.reshape(1, T_OUT))
    return out_flat.reshape(B, N, T_OUT).transpose(0, 2, 1)


# speedup vs baseline: 1.1839x; 1.1839x over previous
import jax
import jax.numpy as jnp
from jax import lax
from jax.experimental import pallas as pl
from jax.experimental.pallas import tpu as pltpu
from jax.experimental.pallas import tpu_sc as plsc

B, T_IN, N, F_IN = 2, 12, 10000, 128
H, GH, T_OUT, E = 128, 128, 24, 160000
BTN = B * T_IN * N

NC, NS, LANES = 2, 16, 16
EPT = E // NS
CHUNK = 128
ROWS = 80
NACC = 10112
SPAN = 632


def _sc_mesh():
    return plsc.VectorSubcoreMesh(core_axis_name="c", subcore_axis_name="s")


HALF = ROWS // 2


def _spmm_body(tbl, srcb, dstb, out, accum, dst_v, srcq, gb0, gb1,
               sem0, sem1):
    c = lax.axis_index("c")
    s = lax.axis_index("s")

    pltpu.sync_copy(dstb.at[s], dst_v)

    def zero_span():
        @pl.loop(0, CHUNK)
        def _(r):
            for k in range(F_IN // LANES):
                gb0[r, pl.ds(k * LANES, LANES)] = jnp.zeros((LANES,),
                                                            jnp.float32)

        for q in range(SPAN // CHUNK):
            pltpu.sync_copy(gb0, accum.at[pl.ds(s * SPAN + q * CHUNK, CHUNK)])
        _rem = SPAN - (SPAN // CHUNK) * CHUNK
        pltpu.sync_copy(gb0.at[pl.ds(0, _rem)],
                        accum.at[pl.ds(s * SPAN + SPAN - _rem, _rem)])

    zero_span()
    plsc.subcore_barrier()

    @pl.loop(0, T_IN)
    def _(t):
        off = (c * T_IN + t) * N

        for half in range(2):
            h0 = half * HALF
            pltpu.sync_copy(srcb.at[s, pl.ds(h0, HALF)], srcq)

            @pl.loop(0, HALF)
            def _(r):
                for k in range(CHUNK // LANES):
                    sl = pl.ds(k * LANES, LANES)
                    srcq[r, sl] = srcq[r, sl] + off

            pltpu.async_copy(tbl.at[srcq.at[0]], gb0, sem0)
            pltpu.async_copy(tbl.at[srcq.at[1]], gb1, sem1)

            @pl.loop(0, HALF, step=2)
            def _(j):
                for b, (gbuf, sem) in enumerate(((gb0, sem0), (gb1, sem1))):
                    ch = j + b
                    pltpu.make_async_copy(tbl.at[srcq.at[ch]], gbuf,
                                          sem).wait()
                    pltpu.sync_copy(gbuf, accum.at[dst_v.at[h0 + ch]],
                                    add=True)

                    @pl.when(ch + 2 < HALF)
                    def _():
                        pltpu.async_copy(tbl.at[srcq.at[ch + 2]], gbuf, sem)

        plsc.subcore_barrier()

        pltpu.sync_copy(accum.at[pl.ds(s * SPAN, 520)],
                        out.at[pl.ds(off + s * SPAN, 520)])

        @pl.when(s < NS - 1)
        def _():
            pltpu.sync_copy(accum.at[pl.ds(s * SPAN + 520, 112)],
                            out.at[pl.ds(off + s * SPAN + 520, 112)])

        zero_span()
        plsc.subcore_barrier()


def _sc_spmm(tbl, srcb, dstb):
    return pl.kernel(
        _spmm_body,
        out_type=jax.ShapeDtypeStruct((BTN, F_IN), jnp.float32),
        mesh=_sc_mesh(),
        scratch_types=[
            pltpu.VMEM_SHARED((NACC, F_IN), jnp.float32),
            pltpu.VMEM((ROWS, CHUNK), jnp.int32),
            pltpu.VMEM((HALF, CHUNK), jnp.int32),
            pltpu.VMEM((CHUNK, F_IN), jnp.float32),
            pltpu.VMEM((CHUNK, F_IN), jnp.float32),
            pltpu.SemaphoreType.DMA,
            pltpu.SemaphoreType.DMA,
        ],
    )(tbl, srcb, dstb)


def _counts_body(dstb, out, cnt_acc, dst_v, buf):
    c = lax.axis_index("c")
    s = lax.axis_index("s")

    @pl.when(c == 0)
    def _():
        def fill(val):
            @pl.loop(0, CHUNK)
            def _(r):
                for k in range(F_IN // LANES):
                    buf[r, pl.ds(k * LANES, LANES)] = jnp.full(
                        (LANES,), val, jnp.float32)

        pltpu.sync_copy(dstb.at[s], dst_v)
        fill(0.0)
        for q in range(SPAN // CHUNK):
            pltpu.sync_copy(buf, cnt_acc.at[pl.ds(s * SPAN + q * CHUNK,
                                                  CHUNK)])
        _rem = SPAN - (SPAN // CHUNK) * CHUNK
        pltpu.sync_copy(buf.at[pl.ds(0, _rem)],
                        cnt_acc.at[pl.ds(s * SPAN + SPAN - _rem, _rem)])
        fill(1.0)
        plsc.subcore_barrier()

        @pl.loop(0, ROWS)
        def _(j):
            pltpu.sync_copy(buf, cnt_acc.at[dst_v.at[j]], add=True)

        plsc.subcore_barrier()
        pltpu.sync_copy(cnt_acc.at[pl.ds(s * SPAN, 520)],
                        out.at[pl.ds(s * SPAN, 520)])

        @pl.when(s < NS - 1)
        def _():
            pltpu.sync_copy(cnt_acc.at[pl.ds(s * SPAN + 520, 112)],
                            out.at[pl.ds(s * SPAN + 520, 112)])


def _sc_counts(dstb):
    return pl.kernel(
        _counts_body,
        out_type=jax.ShapeDtypeStruct((N, F_IN), jnp.float32),
        mesh=_sc_mesh(),
        scratch_types=[
            pltpu.VMEM_SHARED((NACC, F_IN), jnp.float32),
            pltpu.VMEM((ROWS, CHUNK), jnp.int32),
            pltpu.VMEM((CHUNK, F_IN), jnp.float32),
        ],
    )(dstb)


ROWBLK = 1000


def _combine_body(agg_ref, x_ref, cnt_ref, wl_ref, wr_ref, b_ref, o_ref):
    recip = 1.0 / jnp.maximum(cnt_ref[:, 0:1], 1.0)
    agg = agg_ref[...] * recip
    acc = jnp.dot(agg, wl_ref[...], preferred_element_type=jnp.float32)
    acc += jnp.dot(x_ref[...], wr_ref[...], preferred_element_type=jnp.float32)
    o_ref[...] = jnp.maximum(acc + b_ref[...], 0.0)


def _tc_combine(agg, x, cnt, wl_t, wr_t, bias):
    nb = N // ROWBLK
    return pl.pallas_call(
        _combine_body,
        grid=(B * T_IN, nb),
        in_specs=[
            pl.BlockSpec((ROWBLK, F_IN), lambda i, j: (i * nb + j, 0)),
            pl.BlockSpec((ROWBLK, F_IN), lambda i, j: (i * nb + j, 0)),
            pl.BlockSpec((ROWBLK, F_IN), lambda i, j: (j, 0)),
            pl.BlockSpec((F_IN, H), lambda i, j: (0, 0)),
            pl.BlockSpec((F_IN, H), lambda i, j: (0, 0)),
            pl.BlockSpec((1, H), lambda i, j: (0, 0)),
        ],
        out_specs=pl.BlockSpec((ROWBLK, H), lambda i, j: (i * nb + j, 0)),
        out_shape=jax.ShapeDtypeStruct((BTN, H), jnp.float32),
    )(agg, x, cnt, wl_t, wr_t, bias)


GRUBLK = 2000


def _gru_body(h1_ref, wih_ref, bih_ref, whh_ref, bhh_ref, w1_ref, b1_ref,
              w2_ref, b2_ref, o_ref, h_s):
    t = pl.program_id(1)

    @pl.when(t == 0)
    def _():
        h_s[...] = jnp.zeros((GRUBLK, GH), jnp.float32)

    xt = h1_ref[0, 0]
    h = h_s[...]
    gi = jnp.dot(xt, wih_ref[...], preferred_element_type=jnp.float32)
    gi += bih_ref[...]
    gh = jnp.dot(h, whh_ref[...], preferred_element_type=jnp.float32)
    gh += bhh_ref[...]
    r = jax.nn.sigmoid(gi[:, 0:GH] + gh[:, 0:GH])
    z = jax.nn.sigmoid(gi[:, GH:2 * GH] + gh[:, GH:2 * GH])
    n = jnp.tanh(gi[:, 2 * GH:] + r * gh[:, 2 * GH:])
    h = (1.0 - z) * n + z * h
    h_s[...] = h

    @pl.when(t == T_IN - 1)
    def _():
        h1 = jnp.maximum(
            jnp.dot(h, w1_ref[...], preferred_element_type=jnp.float32)
            + b1_ref[...], 0.0)
        o_ref[...] = (
            jnp.dot(h1, w2_ref[...], preferred_element_type=jnp.float32)
            + b2_ref[...])


def _tc_gru(h1_4d, wih_t, b_ih, whh_t, b_hh, w1_t, b1, w2_t, b2):
    nb = N // GRUBLK
    return pl.pallas_call(
        _gru_body,
        grid=(B * nb, T_IN),
        in_specs=[
            pl.BlockSpec((1, 1, GRUBLK, H),
                         lambda r, t: (r // nb, t, r % nb, 0)),
            pl.BlockSpec((H, 3 * GH), lambda r, t: (0, 0)),
            pl.BlockSpec((1, 3 * GH), lambda r, t: (0, 0)),
            pl.BlockSpec((GH, 3 * GH), lambda r, t: (0, 0)),
            pl.BlockSpec((1, 3 * GH), lambda r, t: (0, 0)),
            pl.BlockSpec((GH, GH), lambda r, t: (0, 0)),
            pl.BlockSpec((1, GH), lambda r, t: (0, 0)),
            pl.BlockSpec((GH, T_OUT), lambda r, t: (0, 0)),
            pl.BlockSpec((1, T_OUT), lambda r, t: (0, 0)),
        ],
        out_specs=pl.BlockSpec((GRUBLK, T_OUT), lambda r, t: (r, 0)),
        out_shape=jax.ShapeDtypeStruct((B * N, T_OUT), jnp.float32),
        scratch_shapes=[pltpu.VMEM((GRUBLK, GH), jnp.float32)],
    )(h1_4d, wih_t, b_ih, whh_t, b_hh, w1_t, b1, w2_t, b2)


def kernel(x_seq, edge_index, Wl0, bl0, Wr0, Wl1, bl1, Wr1, W_ih, W_hh,
           b_ih, b_hh, W1, b1, W2, b2):
    src = edge_index[0].astype(jnp.int32)
    dst = edge_index[1].astype(jnp.int32)
    srcb = jnp.pad(src.reshape(NS, EPT), ((0, 0), (0, ROWS * CHUNK - EPT)),
                   constant_values=0).reshape(NS, ROWS, CHUNK)
    dstb = jnp.pad(dst.reshape(NS, EPT), ((0, 0), (0, ROWS * CHUNK - EPT)),
                   constant_values=N).reshape(NS, ROWS, CHUNK)

    x_flat = x_seq.reshape(BTN, F_IN)
    cnt = _sc_counts(dstb)

    agg0 = _sc_spmm(x_flat, srcb, dstb)
    h0 = _tc_combine(agg0, x_flat, cnt, Wl0.T, Wr0.T, bl0.reshape(1, H))
    agg1 = _sc_spmm(h0, srcb, dstb)
    h1 = _tc_combine(agg1, h0, cnt, Wl1.T, Wr1.T, bl1.reshape(1, H))

    out_flat = _tc_gru(h1.reshape(B, T_IN, N, H), W_ih.T,
                       b_ih.reshape(1, 3 * GH), W_hh.T,
                       b_hh.reshape(1, 3 * GH), W1.T, b1.reshape(1, GH),
                       W2.T, b2.reshape(1, T_OUT))
    return out_flat.reshape(B, N, T_OUT).transpose(0, 2, 1)


# speedup vs baseline: 1.2468x; 1.0532x over previous
import functools

import jax
import jax.numpy as jnp
from jax import lax
from jax.experimental import pallas as pl
from jax.experimental.pallas import tpu as pltpu
from jax.experimental.pallas import tpu_sc as plsc

B, T_IN, N, F_IN = 2, 12, 10000, 128
H, GH, T_OUT, E = 128, 128, 24, 160000
BTN = B * T_IN * N

NC, NS, LANES = 2, 16, 16
EPT = E // NS
CHUNK = 128
ROWS = 80
NACC = 10112
SPAN = 632


def _sc_mesh():
    return plsc.VectorSubcoreMesh(core_axis_name="c", subcore_axis_name="s")


HALF = ROWS // 2


def _spmm_body(t0, nt, tt, tbl, srcb, dstb, out, accum, dst_v, srcq, gb0, gb1,
               sem0, sem1):
    c = lax.axis_index("c")
    s = lax.axis_index("s")

    pltpu.sync_copy(dstb.at[s], dst_v)

    def zero_span():
        @pl.loop(0, CHUNK)
        def _(r):
            for k in range(F_IN // LANES):
                gb0[r, pl.ds(k * LANES, LANES)] = jnp.zeros((LANES,),
                                                            jnp.float32)

        for q in range(SPAN // CHUNK):
            pltpu.sync_copy(gb0, accum.at[pl.ds(s * SPAN + q * CHUNK, CHUNK)])
        _rem = SPAN - (SPAN // CHUNK) * CHUNK
        pltpu.sync_copy(gb0.at[pl.ds(0, _rem)],
                        accum.at[pl.ds(s * SPAN + SPAN - _rem, _rem)])

    zero_span()
    plsc.subcore_barrier()

    @pl.loop(0, nt)
    def _(t):
        off = (c * tt + t0 + t) * N
        oout = (c * nt + t) * N

        for half in range(2):
            h0 = half * HALF
            pltpu.sync_copy(srcb.at[s, pl.ds(h0, HALF)], srcq)

            @pl.loop(0, HALF)
            def _(r):
                for k in range(CHUNK // LANES):
                    sl = pl.ds(k * LANES, LANES)
                    srcq[r, sl] = srcq[r, sl] + off

            pltpu.async_copy(tbl.at[srcq.at[0]], gb0, sem0)
            pltpu.async_copy(tbl.at[srcq.at[1]], gb1, sem1)

            @pl.loop(0, HALF, step=2)
            def _(j):
                for b, (gbuf, sem) in enumerate(((gb0, sem0), (gb1, sem1))):
                    ch = j + b
                    pltpu.make_async_copy(tbl.at[srcq.at[ch]], gbuf,
                                          sem).wait()
                    pltpu.sync_copy(gbuf, accum.at[dst_v.at[h0 + ch]],
                                    add=True)

                    @pl.when(ch + 2 < HALF)
                    def _():
                        pltpu.async_copy(tbl.at[srcq.at[ch + 2]], gbuf, sem)

        plsc.subcore_barrier()

        pltpu.sync_copy(accum.at[pl.ds(s * SPAN, 520)],
                        out.at[pl.ds(oout + s * SPAN, 520)])

        @pl.when(s < NS - 1)
        def _():
            pltpu.sync_copy(accum.at[pl.ds(s * SPAN + 520, 112)],
                            out.at[pl.ds(oout + s * SPAN + 520, 112)])

        zero_span()
        plsc.subcore_barrier()


def _sc_spmm(tbl, srcb, dstb, t0, nt, tt):
    return pl.kernel(
        functools.partial(_spmm_body, t0, nt, tt),
        out_type=jax.ShapeDtypeStruct((B * nt * N, F_IN), jnp.float32),
        mesh=_sc_mesh(),
        scratch_types=[
            pltpu.VMEM_SHARED((NACC, F_IN), jnp.float32),
            pltpu.VMEM((ROWS, CHUNK), jnp.int32),
            pltpu.VMEM((HALF, CHUNK), jnp.int32),
            pltpu.VMEM((CHUNK, F_IN), jnp.float32),
            pltpu.VMEM((CHUNK, F_IN), jnp.float32),
            pltpu.SemaphoreType.DMA,
            pltpu.SemaphoreType.DMA,
        ],
    )(tbl, srcb, dstb)


def _counts_body(dstb, out, cnt_acc, dst_v, buf):
    c = lax.axis_index("c")
    s = lax.axis_index("s")

    @pl.when(c == 0)
    def _():
        def fill(val):
            @pl.loop(0, CHUNK)
            def _(r):
                for k in range(F_IN // LANES):
                    buf[r, pl.ds(k * LANES, LANES)] = jnp.full(
                        (LANES,), val, jnp.float32)

        pltpu.sync_copy(dstb.at[s], dst_v)
        fill(0.0)
        for q in range(SPAN // CHUNK):
            pltpu.sync_copy(buf, cnt_acc.at[pl.ds(s * SPAN + q * CHUNK,
                                                  CHUNK)])
        _rem = SPAN - (SPAN // CHUNK) * CHUNK
        pltpu.sync_copy(buf.at[pl.ds(0, _rem)],
                        cnt_acc.at[pl.ds(s * SPAN + SPAN - _rem, _rem)])
        fill(1.0)
        plsc.subcore_barrier()

        @pl.loop(0, ROWS)
        def _(j):
            pltpu.sync_copy(buf, cnt_acc.at[dst_v.at[j]], add=True)

        plsc.subcore_barrier()
        pltpu.sync_copy(cnt_acc.at[pl.ds(s * SPAN, 520)],
                        out.at[pl.ds(s * SPAN, 520)])

        @pl.when(s < NS - 1)
        def _():
            pltpu.sync_copy(cnt_acc.at[pl.ds(s * SPAN + 520, 112)],
                            out.at[pl.ds(s * SPAN + 520, 112)])


def _sc_counts(dstb):
    return pl.kernel(
        _counts_body,
        out_type=jax.ShapeDtypeStruct((N, F_IN), jnp.float32),
        mesh=_sc_mesh(),
        scratch_types=[
            pltpu.VMEM_SHARED((NACC, F_IN), jnp.float32),
            pltpu.VMEM((ROWS, CHUNK), jnp.int32),
            pltpu.VMEM((CHUNK, F_IN), jnp.float32),
        ],
    )(dstb)


ROWBLK = 1000


def _combine_body(agg_ref, x_ref, cnt_ref, wl_ref, wr_ref, b_ref, o_ref):
    recip = 1.0 / jnp.maximum(cnt_ref[:, 0:1], 1.0)
    agg = agg_ref[...] * recip
    acc = jnp.dot(agg, wl_ref[...], preferred_element_type=jnp.float32)
    acc += jnp.dot(x_ref[...], wr_ref[...], preferred_element_type=jnp.float32)
    o_ref[...] = jnp.maximum(acc + b_ref[...], 0.0)


def _tc_combine(agg, x, cnt, wl_t, wr_t, bias, t0, nt, x_tt):
    nb = N // ROWBLK

    def xmap(i, j):
        return ((i // nt * x_tt + t0 + i % nt) * nb + j, 0)

    return pl.pallas_call(
        _combine_body,
        grid=(B * nt, nb),
        in_specs=[
            pl.BlockSpec((ROWBLK, F_IN), lambda i, j: (i * nb + j, 0)),
            pl.BlockSpec((ROWBLK, F_IN), xmap),
            pl.BlockSpec((ROWBLK, F_IN), lambda i, j: (j, 0)),
            pl.BlockSpec((F_IN, H), lambda i, j: (0, 0)),
            pl.BlockSpec((F_IN, H), lambda i, j: (0, 0)),
            pl.BlockSpec((1, H), lambda i, j: (0, 0)),
        ],
        out_specs=pl.BlockSpec((ROWBLK, H), lambda i, j: (i * nb + j, 0)),
        out_shape=jax.ShapeDtypeStruct((B * nt * N, H), jnp.float32),
    )(agg, x, cnt, wl_t, wr_t, bias)


GRUBLK = 2000


def _gru_body(ha_ref, hb_ref, wih_ref, bih_ref, whh_ref, bhh_ref, w1_ref,
              b1_ref, w2_ref, b2_ref, o_ref, h_s):
    t = pl.program_id(1)

    @pl.when(t == 0)
    def _():
        h_s[...] = jnp.zeros((GRUBLK, GH), jnp.float32)

    xt = jnp.where(t < T_IN // 2, ha_ref[0, 0], hb_ref[0, 0])
    h = h_s[...]
    gi = jnp.dot(xt, wih_ref[...], preferred_element_type=jnp.float32)
    gi += bih_ref[...]
    gh = jnp.dot(h, whh_ref[...], preferred_element_type=jnp.float32)
    gh += bhh_ref[...]
    r = jax.nn.sigmoid(gi[:, 0:GH] + gh[:, 0:GH])
    z = jax.nn.sigmoid(gi[:, GH:2 * GH] + gh[:, GH:2 * GH])
    n = jnp.tanh(gi[:, 2 * GH:] + r * gh[:, 2 * GH:])
    h = (1.0 - z) * n + z * h
    h_s[...] = h

    @pl.when(t == T_IN - 1)
    def _():
        h1 = jnp.maximum(
            jnp.dot(h, w1_ref[...], preferred_element_type=jnp.float32)
            + b1_ref[...], 0.0)
        o_ref[...] = (
            jnp.dot(h1, w2_ref[...], preferred_element_type=jnp.float32)
            + b2_ref[...])


def _tc_gru(h1a_4d, h1b_4d, wih_t, b_ih, whh_t, b_hh, w1_t, b1, w2_t, b2):
    nb = N // GRUBLK
    th = T_IN // 2
    return pl.pallas_call(
        _gru_body,
        grid=(B * nb, T_IN),
        in_specs=[
            pl.BlockSpec((1, 1, GRUBLK, H),
                         lambda r, t: (r // nb, t % th, r % nb, 0)),
            pl.BlockSpec((1, 1, GRUBLK, H),
                         lambda r, t: (r // nb, t % th, r % nb, 0)),
            pl.BlockSpec((H, 3 * GH), lambda r, t: (0, 0)),
            pl.BlockSpec((1, 3 * GH), lambda r, t: (0, 0)),
            pl.BlockSpec((GH, 3 * GH), lambda r, t: (0, 0)),
            pl.BlockSpec((1, 3 * GH), lambda r, t: (0, 0)),
            pl.BlockSpec((GH, GH), lambda r, t: (0, 0)),
            pl.BlockSpec((1, GH), lambda r, t: (0, 0)),
            pl.BlockSpec((GH, T_OUT), lambda r, t: (0, 0)),
            pl.BlockSpec((1, T_OUT), lambda r, t: (0, 0)),
        ],
        out_specs=pl.BlockSpec((GRUBLK, T_OUT), lambda r, t: (r, 0)),
        out_shape=jax.ShapeDtypeStruct((B * N, T_OUT), jnp.float32),
        scratch_shapes=[pltpu.VMEM((GRUBLK, GH), jnp.float32)],
    )(h1a_4d, h1b_4d, wih_t, b_ih, whh_t, b_hh, w1_t, b1, w2_t, b2)


def kernel(x_seq, edge_index, Wl0, bl0, Wr0, Wl1, bl1, Wr1, W_ih, W_hh,
           b_ih, b_hh, W1, b1, W2, b2):
    src = edge_index[0].astype(jnp.int32)
    dst = edge_index[1].astype(jnp.int32)
    srcb = jnp.pad(src.reshape(NS, EPT), ((0, 0), (0, ROWS * CHUNK - EPT)),
                   constant_values=0).reshape(NS, ROWS, CHUNK)
    dstb = jnp.pad(dst.reshape(NS, EPT), ((0, 0), (0, ROWS * CHUNK - EPT)),
                   constant_values=N).reshape(NS, ROWS, CHUNK)

    x_flat = x_seq.reshape(BTN, F_IN)
    cnt = _sc_counts(dstb)
    th = T_IN // 2

    wl0, wr0, bb0 = Wl0.T, Wr0.T, bl0.reshape(1, H)
    wl1, wr1, bb1 = Wl1.T, Wr1.T, bl1.reshape(1, H)

    agg0a = _sc_spmm(x_flat, srcb, dstb, 0, th, T_IN)
    agg0b = _sc_spmm(x_flat, srcb, dstb, th, th, T_IN)
    h0a = _tc_combine(agg0a, x_flat, cnt, wl0, wr0, bb0, 0, th, T_IN)
    h0b = _tc_combine(agg0b, x_flat, cnt, wl0, wr0, bb0, th, th, T_IN)
    agg1a = _sc_spmm(h0a, srcb, dstb, 0, th, th)
    agg1b = _sc_spmm(h0b, srcb, dstb, 0, th, th)
    h1a = _tc_combine(agg1a, h0a, cnt, wl1, wr1, bb1, 0, th, th)
    h1b = _tc_combine(agg1b, h0b, cnt, wl1, wr1, bb1, 0, th, th)

    out_flat = _tc_gru(h1a.reshape(B, th, N, H), h1b.reshape(B, th, N, H),
                       W_ih.T, b_ih.reshape(1, 3 * GH), W_hh.T,
                       b_hh.reshape(1, 3 * GH), W1.T, b1.reshape(1, GH),
                       W2.T, b2.reshape(1, T_OUT))
    return out_flat.reshape(B, N, T_OUT).transpose(0, 2, 1)
